# baseline (speedup 1.0000x reference)
import functools

import jax
import jax.numpy as jnp
from jax import lax
from jax.experimental import pallas as pl
from jax.experimental.pallas import tpu as pltpu
from jax.experimental.pallas import tpu_sc as plsc

HIDDEN = 1024
LANES = 16
NTOK = 4 * 8192
NC, NS = 2, 16
NW = NC * NS
TPW = NTOK // NW
CH = 16
NCH = TPW // CH
RINGW = 4
RINGP = 3
LA = 2
GROUP = 12
NMAIN = NCH // GROUP * GROUP


def _emb_body(ids_hbm, pos_hbm, wtab_hbm, ptab_hbm, out_hbm,
              ids_v, pos_v, bufw, bufp, semw, semp, semo):
    c = lax.axis_index("c")
    s = lax.axis_index("s")
    wid = c * NS + s
    base = wid * TPW

    pltpu.sync_copy(ids_hbm.at[pl.ds(base, TPW)], ids_v)
    pltpu.sync_copy(pos_hbm.at[pl.ds(base, TPW)], pos_v)

    def fire_gathers(k, b, q):
        pltpu.async_copy(
            wtab_hbm.at[ids_v.at[pl.ds(k * CH, CH)]], bufw.at[b], semw[b])
        pltpu.async_copy(
            ptab_hbm.at[pos_v.at[pl.ds(k * CH, CH)]], bufp.at[q], semp[q])

    def wait_gathers(k, b, q):
        pltpu.make_async_copy(
            wtab_hbm.at[ids_v.at[pl.ds(k * CH, CH)]], bufw.at[b],
            semw[b]).wait()
        pltpu.make_async_copy(
            ptab_hbm.at[pos_v.at[pl.ds(k * CH, CH)]], bufp.at[q],
            semp[q]).wait()

    def fire_scatter(k, b):
        pltpu.async_copy(
            bufw.at[b], out_hbm.at[pl.ds(base + k * CH, CH)], semo[b])

    def wait_scatter(k, b):
        pltpu.make_async_copy(
            bufw.at[b], out_hbm.at[pl.ds(base + k * CH, CH)], semo[b]).wait()

    def add_rows(b, q):
        @pl.loop(0, CH)
        def _row(t):
            for i in range(HIDDEN // LANES):
                sl = pl.ds(i * LANES, LANES)
                plsc.addupdate(bufw.at[b, t, sl], bufp[q, t, sl])

    for k in range(LA):
        fire_gathers(k, k % RINGW, k % RINGP)

    @pl.loop(0, NMAIN, step=GROUP)
    def _group(g):
        for j in range(GROUP):
            k = g + j
            b = j % RINGW
            q = j % RINGP
            wait_gathers(k, b, q)
            nk = k + LA
            nb = (j + LA) % RINGW
            nq = (j + LA) % RINGP

            @pl.when(nk < NCH)
            def _prefetch():
                @pl.when(k >= RINGW - LA)
                def _drain():
                    wait_scatter(k - (RINGW - LA), nb)
                fire_gathers(nk, nb, nq)

            add_rows(b, q)
            fire_scatter(k, b)

    for k in range(NMAIN, NCH):
        wait_gathers(k, k % RINGW, k % RINGP)
        if k + LA < NCH:
            wait_scatter(k - (RINGW - LA), (k + LA) % RINGW)
            fire_gathers(k + LA, (k + LA) % RINGW, (k + LA) % RINGP)
        add_rows(k % RINGW, k % RINGP)
        fire_scatter(k, k % RINGW)
    for k in range(max(0, NCH - RINGW), NCH):
        wait_scatter(k, k % RINGW)


@functools.partial(
    pl.kernel,
    out_type=jax.ShapeDtypeStruct((NTOK, HIDDEN), jnp.float32),
    mesh=plsc.VectorSubcoreMesh(core_axis_name="c", subcore_axis_name="s"),
    scratch_types=[
        pltpu.VMEM((TPW,), jnp.int32),
        pltpu.VMEM((TPW,), jnp.int32),
        pltpu.VMEM((RINGW, CH, HIDDEN), jnp.float32),
        pltpu.VMEM((RINGP, CH, HIDDEN), jnp.float32),
        [pltpu.SemaphoreType.DMA] * RINGW,
        [pltpu.SemaphoreType.DMA] * RINGP,
        [pltpu.SemaphoreType.DMA] * RINGW,
    ],
)
def _emb_call(ids_hbm, pos_hbm, wtab_hbm, ptab_hbm, out_hbm,
              ids_v, pos_v, bufw, bufp, semw, semp, semo):
    _emb_body(ids_hbm, pos_hbm, wtab_hbm, ptab_hbm, out_hbm,
              ids_v, pos_v, bufw, bufp, semw, semp, semo)


@jax.jit
def kernel(input_ids, position_ids, word_table, pos_table):
    bsh = input_ids.shape
    ids = input_ids.reshape(-1).astype(jnp.int32)
    pos = position_ids.reshape(-1).astype(jnp.int32)
    out = _emb_call(ids, pos, word_table, pos_table)
    return out.reshape(*bsh, HIDDEN)


# speedup vs baseline: 1.5134x; 1.5134x over previous
import functools

import jax
import jax.numpy as jnp
from jax import lax
from jax.experimental import pallas as pl
from jax.experimental.pallas import tpu as pltpu
from jax.experimental.pallas import tpu_sc as plsc

HIDDEN = 1024
LANES = 16
NTOK = 4 * 8192
NC, NS = 2, 16
NW = NC * NS
TPW = NTOK // NW
CH = 8
NCH = TPW // CH
RING = 6
LA = 4
NMAIN = NCH // RING * RING


def _emb_body(ids_hbm, pos_hbm, wtab_hbm, ptab_hbm, out_hbm,
              ids_v, pos_v, bufw, bufp, semw, semp, semo):
    c = lax.axis_index("c")
    s = lax.axis_index("s")
    wid = c * NS + s
    base = wid * TPW

    pltpu.sync_copy(ids_hbm.at[pl.ds(base, TPW)], ids_v)
    pltpu.sync_copy(pos_hbm.at[pl.ds(base, TPW)], pos_v)

    def fire_gathers(k, b):
        pltpu.async_copy(
            wtab_hbm.at[ids_v.at[pl.ds(k * CH, CH)]], bufw.at[b], semw[b])
        pltpu.async_copy(
            ptab_hbm.at[pos_v.at[pl.ds(k * CH, CH)]], bufp.at[b], semp[b])

    def wait_gathers(k, b):
        pltpu.make_async_copy(
            wtab_hbm.at[ids_v.at[pl.ds(k * CH, CH)]], bufw.at[b],
            semw[b]).wait()
        pltpu.make_async_copy(
            ptab_hbm.at[pos_v.at[pl.ds(k * CH, CH)]], bufp.at[b],
            semp[b]).wait()

    def fire_scatter(k, b):
        pltpu.async_copy(
            bufw.at[b], out_hbm.at[pl.ds(base + k * CH, CH)], semo[b])

    def wait_scatter(k, b):
        pltpu.make_async_copy(
            bufw.at[b], out_hbm.at[pl.ds(base + k * CH, CH)], semo[b]).wait()

    def add_rows(b):
        @pl.loop(0, CH)
        def _row(t):
            for i in range(HIDDEN // LANES):
                sl = pl.ds(i * LANES, LANES)
                plsc.addupdate(bufw.at[b, t, sl], bufp[b, t, sl])

    for k in range(LA):
        fire_gathers(k, k)

    @pl.loop(0, NMAIN, step=RING)
    def _group(g):
        for b in range(RING):
            k = g + b
            wait_gathers(k, b)
            nk = k + LA
            nb = (b + LA) % RING

            @pl.when(nk < NCH)
            def _prefetch():
                @pl.when(k >= RING - LA)
                def _drain():
                    wait_scatter(k - (RING - LA), nb)
                fire_gathers(nk, nb)

            add_rows(b)
            fire_scatter(k, b)

    for k in range(NMAIN, NCH):
        b = k % RING
        wait_gathers(k, b)
        add_rows(b)
        fire_scatter(k, b)
    for k in range(max(0, NCH - RING), NCH):
        wait_scatter(k, k % RING)


@functools.partial(
    pl.kernel,
    out_type=jax.ShapeDtypeStruct((NTOK, HIDDEN), jnp.float32),
    mesh=plsc.VectorSubcoreMesh(core_axis_name="c", subcore_axis_name="s"),
    scratch_types=[
        pltpu.VMEM((TPW,), jnp.int32),
        pltpu.VMEM((TPW,), jnp.int32),
        pltpu.VMEM((RING, CH, HIDDEN), jnp.float32),
        pltpu.VMEM((RING, CH, HIDDEN), jnp.float32),
        [pltpu.SemaphoreType.DMA] * RING,
        [pltpu.SemaphoreType.DMA] * RING,
        [pltpu.SemaphoreType.DMA] * RING,
    ],
)
def _emb_call(ids_hbm, pos_hbm, wtab_hbm, ptab_hbm, out_hbm,
              ids_v, pos_v, bufw, bufp, semw, semp, semo):
    _emb_body(ids_hbm, pos_hbm, wtab_hbm, ptab_hbm, out_hbm,
              ids_v, pos_v, bufw, bufp, semw, semp, semo)


@jax.jit
def kernel(input_ids, position_ids, word_table, pos_table):
    bsh = input_ids.shape
    ids = input_ids.reshape(-1).astype(jnp.int32)
    pos = position_ids.reshape(-1).astype(jnp.int32)
    out = _emb_call(ids, pos, word_table, pos_table)
    return out.reshape(*bsh, HIDDEN)


# speedup vs baseline: 1.5220x; 1.0057x over previous
import functools

import jax
import jax.numpy as jnp
from jax import lax
from jax.experimental import pallas as pl
from jax.experimental.pallas import tpu as pltpu
from jax.experimental.pallas import tpu_sc as plsc

HIDDEN = 1024
LANES = 16
NTOK = 4 * 8192
NC, NS = 2, 16
NW = NC * NS
TPW = NTOK // NW
CH = 8
NCH = TPW // CH
RING = 6
LA = 3
NMAIN = NCH // RING * RING


def _emb_body(ids_hbm, pos_hbm, wtab_hbm, ptab_hbm, out_hbm,
              ids_v, pos_v, bufw, bufp, semw, semp, semo):
    c = lax.axis_index("c")
    s = lax.axis_index("s")
    wid = c * NS + s
    base = wid * TPW

    pltpu.sync_copy(ids_hbm.at[pl.ds(base, TPW)], ids_v)
    pltpu.sync_copy(pos_hbm.at[pl.ds(base, TPW)], pos_v)

    def fire_gathers(k, b):
        pltpu.async_copy(
            wtab_hbm.at[ids_v.at[pl.ds(k * CH, CH)]], bufw.at[b], semw[b])
        pltpu.async_copy(
            ptab_hbm.at[pos_v.at[pl.ds(k * CH, CH)]], bufp.at[b], semp[b])

    def wait_gathers(k, b):
        pltpu.make_async_copy(
            wtab_hbm.at[ids_v.at[pl.ds(k * CH, CH)]], bufw.at[b],
            semw[b]).wait()
        pltpu.make_async_copy(
            ptab_hbm.at[pos_v.at[pl.ds(k * CH, CH)]], bufp.at[b],
            semp[b]).wait()

    def fire_scatter(k, b):
        pltpu.async_copy(
            bufw.at[b], out_hbm.at[pl.ds(base + k * CH, CH)], semo[b])

    def wait_scatter(k, b):
        pltpu.make_async_copy(
            bufw.at[b], out_hbm.at[pl.ds(base + k * CH, CH)], semo[b]).wait()

    def add_rows(b):
        @pl.loop(0, CH)
        def _row(t):
            for i in range(HIDDEN // LANES):
                sl = pl.ds(i * LANES, LANES)
                plsc.addupdate(bufw.at[b, t, sl], bufp[b, t, sl])

    for k in range(LA):
        fire_gathers(k, k)

    @pl.loop(0, NMAIN, step=RING)
    def _group(g):
        for b in range(RING):
            k = g + b
            wait_gathers(k, b)
            nk = k + LA
            nb = (b + LA) % RING

            @pl.when(nk < NCH)
            def _prefetch():
                @pl.when(k >= RING - LA)
                def _drain():
                    wait_scatter(k - (RING - LA), nb)
                fire_gathers(nk, nb)

            add_rows(b)
            fire_scatter(k, b)

    for k in range(NMAIN, NCH):
        b = k % RING
        wait_gathers(k, b)
        add_rows(b)
        fire_scatter(k, b)
    for k in range(max(0, NCH - RING), NCH):
        wait_scatter(k, k % RING)


@functools.partial(
    pl.kernel,
    out_type=jax.ShapeDtypeStruct((NTOK, HIDDEN), jnp.float32),
    mesh=plsc.VectorSubcoreMesh(core_axis_name="c", subcore_axis_name="s"),
    scratch_types=[
        pltpu.VMEM((TPW,), jnp.int32),
        pltpu.VMEM((TPW,), jnp.int32),
        pltpu.VMEM((RING, CH, HIDDEN), jnp.float32),
        pltpu.VMEM((RING, CH, HIDDEN), jnp.float32),
        [pltpu.SemaphoreType.DMA] * RING,
        [pltpu.SemaphoreType.DMA] * RING,
        [pltpu.SemaphoreType.DMA] * RING,
    ],
)
def _emb_call(ids_hbm, pos_hbm, wtab_hbm, ptab_hbm, out_hbm,
              ids_v, pos_v, bufw, bufp, semw, semp, semo):
    _emb_body(ids_hbm, pos_hbm, wtab_hbm, ptab_hbm, out_hbm,
              ids_v, pos_v, bufw, bufp, semw, semp, semo)


@jax.jit
def kernel(input_ids, position_ids, word_table, pos_table):
    bsh = input_ids.shape
    ids = input_ids.reshape(-1).astype(jnp.int32)
    pos = position_ids.reshape(-1).astype(jnp.int32)
    out = _emb_call(ids, pos, word_table, pos_table)
    return out.reshape(*bsh, HIDDEN)
